# Initial kernel scaffold; baseline (speedup 1.0000x reference)
import jax
import jax.numpy as jnp
from jax.experimental import pallas as pl


def kernel(x, edge_index, e, xbatch, params):
    raise NotImplementedError("write your pallas kernel here")


# speedup vs baseline: 1.1009x; 1.1009x over previous
import jax
import jax.numpy as jnp
from jax.experimental import pallas as pl

BE = 6400
BN = 2000


def _leaky(v, s):
    return jnp.where(v >= 0, v, s * v)


def _ec_body(dst_ref, src_ref, wa_ref, ba_ref, wb_ref, bb_ref, o_ref):
    d = dst_ref[...]
    s = src_ref[...]
    h = jnp.concatenate([d, s - d], axis=1)
    h = _leaky(jnp.dot(h, wa_ref[...], preferred_element_type=jnp.float32)
               + ba_ref[...], 0.1)
    h = _leaky(jnp.dot(h, wb_ref[...], preferred_element_type=jnp.float32)
               + bb_ref[...], 0.1)
    o_ref[...] = h


def _ec_mlp(dst, src, pa, pb):
    E0, F = dst.shape
    Fm = pa[0].shape[1]
    Fo = pb[0].shape[1]
    return pl.pallas_call(
        _ec_body,
        grid=(E0 // BE,),
        in_specs=[
            pl.BlockSpec((BE, F), lambda i: (i, 0)),
            pl.BlockSpec((BE, F), lambda i: (i, 0)),
            pl.BlockSpec((2 * F, Fm), lambda i: (0, 0)),
            pl.BlockSpec((1, Fm), lambda i: (0, 0)),
            pl.BlockSpec((Fm, Fo), lambda i: (0, 0)),
            pl.BlockSpec((1, Fo), lambda i: (0, 0)),
        ],
        out_specs=pl.BlockSpec((BE, Fo), lambda i: (i, 0)),
        out_shape=jax.ShapeDtypeStruct((E0, Fo), jnp.float32),
    )(dst, src, pa[0], pa[1][None], pb[0], pb[1][None])


def _meta_body(hr_ref, hc_ref, wea_ref, bea_ref, web_ref, beb_ref,
               wna_ref, bna_ref, wnb_ref, bnb_ref, o_ref):
    hr = hr_ref[...]
    hc = hc_ref[...]
    ea = jnp.concatenate([hr, hc], axis=1)
    ea = _leaky(jnp.dot(ea, wea_ref[...], preferred_element_type=jnp.float32)
                + bea_ref[...], 0.12)
    ea = jnp.dot(ea, web_ref[...], preferred_element_type=jnp.float32) + beb_ref[...]
    out = jnp.concatenate([hc, ea], axis=1)
    out = _leaky(jnp.dot(out, wna_ref[...], preferred_element_type=jnp.float32)
                 + bna_ref[...], 0.12)
    out = jnp.dot(out, wnb_ref[...], preferred_element_type=jnp.float32) + bnb_ref[...]
    o_ref[...] = out


def _meta_mlp(hrow, hcol, p):
    E0 = hrow.shape[0]
    return pl.pallas_call(
        _meta_body,
        grid=(E0 // BE,),
        in_specs=[
            pl.BlockSpec((BE, 64), lambda i: (i, 0)),
            pl.BlockSpec((BE, 64), lambda i: (i, 0)),
            pl.BlockSpec((128, 64), lambda i: (0, 0)),
            pl.BlockSpec((1, 64), lambda i: (0, 0)),
            pl.BlockSpec((64, 16), lambda i: (0, 0)),
            pl.BlockSpec((1, 16), lambda i: (0, 0)),
            pl.BlockSpec((80, 64), lambda i: (0, 0)),
            pl.BlockSpec((1, 64), lambda i: (0, 0)),
            pl.BlockSpec((64, 32), lambda i: (0, 0)),
            pl.BlockSpec((1, 32), lambda i: (0, 0)),
        ],
        out_specs=pl.BlockSpec((BE, 32), lambda i: (i, 0)),
        out_shape=jax.ShapeDtypeStruct((E0, 32), jnp.float32),
    )(hrow, hcol, p['ema'][0], p['ema'][1][None], p['emb'][0], p['emb'][1][None],
      p['n1a'][0], p['n1a'][1][None], p['n1b'][0], p['n1b'][1][None])


def _final_body(sum_ref, cnt_ref, wa_ref, ba_ref, wb_ref, bb_ref, o_ref):
    s = sum_ref[...]
    c = cnt_ref[...]
    mean = s / jnp.maximum(c, 1.0)
    z = _leaky(jnp.dot(mean, wa_ref[...], preferred_element_type=jnp.float32)
               + ba_ref[...], 0.12)
    z = jnp.dot(z, wb_ref[...], preferred_element_type=jnp.float32) + bb_ref[...]
    m = jnp.max(z, axis=1, keepdims=True)
    lse = m + jnp.log(jnp.sum(jnp.exp(z - m), axis=1, keepdims=True))
    o_ref[...] = z - lse


def _final_mlp(summed, cnt, p):
    N0 = summed.shape[0]
    return pl.pallas_call(
        _final_body,
        grid=(N0 // BN,),
        in_specs=[
            pl.BlockSpec((BN, 32), lambda i: (i, 0)),
            pl.BlockSpec((BN, 1), lambda i: (i, 0)),
            pl.BlockSpec((32, 16), lambda i: (0, 0)),
            pl.BlockSpec((1, 16), lambda i: (0, 0)),
            pl.BlockSpec((16, 2), lambda i: (0, 0)),
            pl.BlockSpec((1, 2), lambda i: (0, 0)),
        ],
        out_specs=pl.BlockSpec((BN, 2), lambda i: (i, 0)),
        out_shape=jax.ShapeDtypeStruct((N0, 2), jnp.float32),
    )(summed, cnt, p['n2a'][0], p['n2a'][1][None], p['n2b'][0], p['n2b'][1][None])


def _edge_conv(h, row, col, pa, pb, n):
    dst = h[col]
    src = h[row]
    out = _ec_mlp(dst, src, pa, pb)
    out = jax.ops.segment_max(out, col, num_segments=n)
    return jnp.where(jnp.isfinite(out), out, 0.0)


def kernel(x, edge_index, e, xbatch, params):
    p = params
    n = x.shape[0]
    row, col = edge_index[0], edge_index[1]
    h = _edge_conv(x, row, col, p['ec1a'], p['ec1b'], n)
    h = _edge_conv(h, row, col, p['ec2a'], p['ec2b'], n)
    h = _edge_conv(h, row, col, p['ec3a'], p['ec3b'], n)
    out = _meta_mlp(h[row], h[col], p)
    summed = jax.ops.segment_sum(out, row, num_segments=n)
    cnt = jax.ops.segment_sum(jnp.ones((row.shape[0],), jnp.float32), row,
                              num_segments=n)
    return _final_mlp(summed, cnt[:, None], p)


# speedup vs baseline: 1.8292x; 1.6615x over previous
import jax
import jax.numpy as jnp
from jax import lax
from jax.experimental import pallas as pl
from jax.experimental.pallas import tpu as pltpu
from jax.experimental.pallas import tpu_sc as plsc

BE = 6400
BN = 2000

_NC, _NS = 2, 16
_NW = _NC * _NS
_CH = 128
_NBUF = 4


def _sc_gather(table, idx):
    B = idx.shape[0]
    F = table.shape[1]
    n_chunks = B // _CH
    q, r = divmod(n_chunks, _NW)
    M = q // _NBUF
    idx_p = jnp.concatenate([idx, jnp.zeros((_CH,), jnp.int32)])

    def body(table_hbm, idx_hbm, out_hbm, idx_v, bufs, sems):
        wid = lax.axis_index("s") * _NC + lax.axis_index("c")
        n_w = q + jnp.where(wid < r, 1, 0)
        first = q * wid + jnp.minimum(wid, r)
        pltpu.sync_copy(idx_hbm.at[pl.ds(first * _CH, (q + 1) * _CH)],
                        idx_v.at[pl.ds(0, (q + 1) * _CH)])

        def start(c, b):
            pltpu.async_copy(
                table_hbm.at[idx_v.at[pl.ds(c * _CH, _CH)]], bufs[b], sems[b])

        def drain(c, b):
            pltpu.make_async_copy(
                table_hbm.at[idx_v.at[pl.ds(c * _CH, _CH)]], bufs[b],
                sems[b]).wait()
            pltpu.sync_copy(bufs[b], out_hbm.at[pl.ds((first + c) * _CH, _CH)])

        for b in range(_NBUF):
            start(b, b)

        def loop(g, carry):
            for b in range(_NBUF):
                c = g * _NBUF + b
                drain(c, b)
                nxt = c + _NBUF

                @pl.when(nxt < n_w)
                def _():
                    start(nxt, b)

            return carry

        lax.fori_loop(0, M, loop, 0)
        for b in range(_NBUF):
            c = M * _NBUF + b

            @pl.when(c < n_w)
            def _():
                drain(c, b)

    return pl.kernel(
        body,
        out_type=jax.ShapeDtypeStruct((B, F), jnp.float32),
        mesh=plsc.VectorSubcoreMesh(core_axis_name="c", subcore_axis_name="s"),
        compiler_params=pltpu.CompilerParams(use_tc_tiling_on_sc=False),
        scratch_types=[
            pltpu.VMEM(((q + 2) * _CH,), jnp.int32),
            [pltpu.VMEM((_CH, F), jnp.float32) for _ in range(_NBUF)],
            [pltpu.SemaphoreType.DMA for _ in range(_NBUF)],
        ],
    )(table, idx_p)


def _leaky(v, s):
    return jnp.where(v >= 0, v, s * v)


def _ec_body(dst_ref, src_ref, wa_ref, ba_ref, wb_ref, bb_ref, o_ref):
    d = dst_ref[...]
    s = src_ref[...]
    h = jnp.concatenate([d, s - d], axis=1)
    h = _leaky(jnp.dot(h, wa_ref[...], preferred_element_type=jnp.float32)
               + ba_ref[...], 0.1)
    h = _leaky(jnp.dot(h, wb_ref[...], preferred_element_type=jnp.float32)
               + bb_ref[...], 0.1)
    o_ref[...] = h


def _ec_mlp(g, pa, pb):
    E0 = g.shape[0] // 2
    F = g.shape[1]
    Fm = pa[0].shape[1]
    Fo = pb[0].shape[1]
    eoff = E0 // BE
    return pl.pallas_call(
        _ec_body,
        grid=(E0 // BE,),
        in_specs=[
            pl.BlockSpec((BE, F), lambda i: (i + eoff, 0)),
            pl.BlockSpec((BE, F), lambda i: (i, 0)),
            pl.BlockSpec((2 * F, Fm), lambda i: (0, 0)),
            pl.BlockSpec((1, Fm), lambda i: (0, 0)),
            pl.BlockSpec((Fm, Fo), lambda i: (0, 0)),
            pl.BlockSpec((1, Fo), lambda i: (0, 0)),
        ],
        out_specs=pl.BlockSpec((BE, Fo), lambda i: (i, 0)),
        out_shape=jax.ShapeDtypeStruct((E0, Fo), jnp.float32),
    )(g, g, pa[0], pa[1][None], pb[0], pb[1][None])


def _meta_body(hr_ref, hc_ref, wea_ref, bea_ref, web_ref, beb_ref,
               wna_ref, bna_ref, wnb_ref, bnb_ref, o_ref):
    hr = hr_ref[...]
    hc = hc_ref[...]
    ea = jnp.concatenate([hr, hc], axis=1)
    ea = _leaky(jnp.dot(ea, wea_ref[...], preferred_element_type=jnp.float32)
                + bea_ref[...], 0.12)
    ea = jnp.dot(ea, web_ref[...], preferred_element_type=jnp.float32) + beb_ref[...]
    out = jnp.concatenate([hc, ea], axis=1)
    out = _leaky(jnp.dot(out, wna_ref[...], preferred_element_type=jnp.float32)
                 + bna_ref[...], 0.12)
    out = jnp.dot(out, wnb_ref[...], preferred_element_type=jnp.float32) + bnb_ref[...]
    o_ref[...] = out


def _meta_mlp(g, p):
    E0 = g.shape[0] // 2
    eoff = E0 // BE
    return pl.pallas_call(
        _meta_body,
        grid=(E0 // BE,),
        in_specs=[
            pl.BlockSpec((BE, 64), lambda i: (i, 0)),
            pl.BlockSpec((BE, 64), lambda i: (i + eoff, 0)),
            pl.BlockSpec((128, 64), lambda i: (0, 0)),
            pl.BlockSpec((1, 64), lambda i: (0, 0)),
            pl.BlockSpec((64, 16), lambda i: (0, 0)),
            pl.BlockSpec((1, 16), lambda i: (0, 0)),
            pl.BlockSpec((80, 64), lambda i: (0, 0)),
            pl.BlockSpec((1, 64), lambda i: (0, 0)),
            pl.BlockSpec((64, 32), lambda i: (0, 0)),
            pl.BlockSpec((1, 32), lambda i: (0, 0)),
        ],
        out_specs=pl.BlockSpec((BE, 32), lambda i: (i, 0)),
        out_shape=jax.ShapeDtypeStruct((E0, 32), jnp.float32),
    )(g, g, p['ema'][0], p['ema'][1][None], p['emb'][0], p['emb'][1][None],
      p['n1a'][0], p['n1a'][1][None], p['n1b'][0], p['n1b'][1][None])


def _final_body(sum_ref, cnt_ref, wa_ref, ba_ref, wb_ref, bb_ref, o_ref):
    s = sum_ref[...]
    c = cnt_ref[...]
    mean = s / jnp.maximum(c, 1.0)
    z = _leaky(jnp.dot(mean, wa_ref[...], preferred_element_type=jnp.float32)
               + ba_ref[...], 0.12)
    z = jnp.dot(z, wb_ref[...], preferred_element_type=jnp.float32) + bb_ref[...]
    m = jnp.max(z, axis=1, keepdims=True)
    lse = m + jnp.log(jnp.sum(jnp.exp(z - m), axis=1, keepdims=True))
    o_ref[...] = z - lse


def _final_mlp(summed, cnt, p):
    N0 = summed.shape[0]
    return pl.pallas_call(
        _final_body,
        grid=(N0 // BN,),
        in_specs=[
            pl.BlockSpec((BN, 32), lambda i: (i, 0)),
            pl.BlockSpec((BN, 1), lambda i: (i, 0)),
            pl.BlockSpec((32, 16), lambda i: (0, 0)),
            pl.BlockSpec((1, 16), lambda i: (0, 0)),
            pl.BlockSpec((16, 2), lambda i: (0, 0)),
            pl.BlockSpec((1, 2), lambda i: (0, 0)),
        ],
        out_specs=pl.BlockSpec((BN, 2), lambda i: (i, 0)),
        out_shape=jax.ShapeDtypeStruct((N0, 2), jnp.float32),
    )(summed, cnt, p['n2a'][0], p['n2a'][1][None], p['n2b'][0], p['n2b'][1][None])


def _edge_conv(h, ei_flat, col, pa, pb, n):
    g = _sc_gather(h, ei_flat)
    out = _ec_mlp(g, pa, pb)
    out = jax.ops.segment_max(out, col, num_segments=n)
    return jnp.where(jnp.isfinite(out), out, 0.0)


def kernel(x, edge_index, e, xbatch, params):
    p = params
    n = x.shape[0]
    row, col = edge_index[0], edge_index[1]
    ei_flat = edge_index.reshape(-1)
    h = _edge_conv(x, ei_flat, col, p['ec1a'], p['ec1b'], n)
    h = _edge_conv(h, ei_flat, col, p['ec2a'], p['ec2b'], n)
    h = _edge_conv(h, ei_flat, col, p['ec3a'], p['ec3b'], n)
    out = _meta_mlp(_sc_gather(h, ei_flat), p)
    summed = jax.ops.segment_sum(out, row, num_segments=n)
    cnt = jax.ops.segment_sum(jnp.ones((row.shape[0],), jnp.float32), row,
                              num_segments=n)
    return _final_mlp(summed, cnt[:, None], p)


# speedup vs baseline: 2.0234x; 1.1062x over previous
import functools

import jax
import jax.numpy as jnp
from jax import lax
from jax.experimental import pallas as pl
from jax.experimental.pallas import tpu as pltpu
from jax.experimental.pallas import tpu_sc as plsc

BE = 6400
BN = 2000

_NC, _NS = 2, 16
_NW = _NC * _NS
_CH = 128
_NBUF = 3


def _sc_gather(table, idx, F):
    B = idx.shape[0]
    n_chunks = B // _CH
    q, r = divmod(n_chunks, _NW)
    M = q // _NBUF
    idx_p = jnp.concatenate([idx, jnp.zeros((_CH,), jnp.int32)])

    def body(table_hbm, idx_hbm, out_hbm, idx_v, bufs, sems):
        wid = lax.axis_index("s") * _NC + lax.axis_index("c")
        n_w = q + jnp.where(wid < r, 1, 0)
        first = q * wid + jnp.minimum(wid, r)
        pltpu.sync_copy(idx_hbm.at[pl.ds(first * _CH, (q + 1) * _CH)],
                        idx_v.at[pl.ds(0, (q + 1) * _CH)])

        def start(c, b):
            pltpu.async_copy(
                table_hbm.at[idx_v.at[pl.ds(c * _CH, _CH)]], bufs[b], sems[b])

        def drain(c, b):
            pltpu.make_async_copy(
                table_hbm.at[idx_v.at[pl.ds(c * _CH, _CH)]], bufs[b],
                sems[b]).wait()
            pltpu.sync_copy(bufs[b], out_hbm.at[pl.ds((first + c) * _CH, _CH)])

        for b in range(_NBUF):
            start(b, b)

        def loop(g, carry):
            for b in range(_NBUF):
                c = g * _NBUF + b
                drain(c, b)
                nxt = c + _NBUF

                @pl.when(nxt < n_w)
                def _():
                    start(nxt, b)

            return carry

        lax.fori_loop(0, M, loop, 0)
        for b in range(_NBUF):
            c = M * _NBUF + b

            @pl.when(c < n_w)
            def _():
                drain(c, b)

    return pl.kernel(
        body,
        out_type=jax.ShapeDtypeStruct((B, 128), jnp.float32),
        mesh=plsc.VectorSubcoreMesh(core_axis_name="c", subcore_axis_name="s"),
        scratch_types=[
            pltpu.VMEM(((q + 1 + _NBUF) * _CH,), jnp.int32),
            [pltpu.VMEM((_CH, 128), jnp.float32) for _ in range(_NBUF)],
            [pltpu.SemaphoreType.DMA for _ in range(_NBUF)],
        ],
    )(table, idx_p)


def _leaky(v, s):
    return jnp.where(v >= 0, v, s * v)


def _ec_body(F, dst_ref, src_ref, wa_ref, ba_ref, wb_ref, bb_ref, o_ref):
    d = dst_ref[:, :F]
    s = src_ref[:, :F]
    h = jnp.concatenate([d, s - d], axis=1)
    h = _leaky(jnp.dot(h, wa_ref[...], preferred_element_type=jnp.float32)
               + ba_ref[...], 0.1)
    h = _leaky(jnp.dot(h, wb_ref[...], preferred_element_type=jnp.float32)
               + bb_ref[...], 0.1)
    o_ref[...] = h


def _ec_mlp(g, F, pa, pb):
    E0 = g.shape[0] // 2
    Fm = pa[0].shape[1]
    Fo = pb[0].shape[1]
    eoff = E0 // BE
    return pl.pallas_call(
        functools.partial(_ec_body, F),
        grid=(E0 // BE,),
        in_specs=[
            pl.BlockSpec((BE, 128), lambda i: (i + eoff, 0)),
            pl.BlockSpec((BE, 128), lambda i: (i, 0)),
            pl.BlockSpec((2 * F, Fm), lambda i: (0, 0)),
            pl.BlockSpec((1, Fm), lambda i: (0, 0)),
            pl.BlockSpec((Fm, Fo), lambda i: (0, 0)),
            pl.BlockSpec((1, Fo), lambda i: (0, 0)),
        ],
        out_specs=pl.BlockSpec((BE, Fo), lambda i: (i, 0)),
        out_shape=jax.ShapeDtypeStruct((E0, Fo), jnp.float32),
    )(g, g, pa[0], pa[1][None], pb[0], pb[1][None])


def _meta_body(hr_ref, hc_ref, wea_ref, bea_ref, web_ref, beb_ref,
               wna_ref, bna_ref, wnb_ref, bnb_ref, o_ref):
    hr = hr_ref[:, :64]
    hc = hc_ref[:, :64]
    ea = jnp.concatenate([hr, hc], axis=1)
    ea = _leaky(jnp.dot(ea, wea_ref[...], preferred_element_type=jnp.float32)
                + bea_ref[...], 0.12)
    ea = jnp.dot(ea, web_ref[...], preferred_element_type=jnp.float32) + beb_ref[...]
    out = jnp.concatenate([hc, ea], axis=1)
    out = _leaky(jnp.dot(out, wna_ref[...], preferred_element_type=jnp.float32)
                 + bna_ref[...], 0.12)
    out = jnp.dot(out, wnb_ref[...], preferred_element_type=jnp.float32) + bnb_ref[...]
    o_ref[...] = out


def _meta_mlp(g, p):
    E0 = g.shape[0] // 2
    eoff = E0 // BE
    return pl.pallas_call(
        _meta_body,
        grid=(E0 // BE,),
        in_specs=[
            pl.BlockSpec((BE, 128), lambda i: (i, 0)),
            pl.BlockSpec((BE, 128), lambda i: (i + eoff, 0)),
            pl.BlockSpec((128, 64), lambda i: (0, 0)),
            pl.BlockSpec((1, 64), lambda i: (0, 0)),
            pl.BlockSpec((64, 16), lambda i: (0, 0)),
            pl.BlockSpec((1, 16), lambda i: (0, 0)),
            pl.BlockSpec((80, 64), lambda i: (0, 0)),
            pl.BlockSpec((1, 64), lambda i: (0, 0)),
            pl.BlockSpec((64, 32), lambda i: (0, 0)),
            pl.BlockSpec((1, 32), lambda i: (0, 0)),
        ],
        out_specs=pl.BlockSpec((BE, 32), lambda i: (i, 0)),
        out_shape=jax.ShapeDtypeStruct((E0, 32), jnp.float32),
    )(g, g, p['ema'][0], p['ema'][1][None], p['emb'][0], p['emb'][1][None],
      p['n1a'][0], p['n1a'][1][None], p['n1b'][0], p['n1b'][1][None])


def _final_body(sum_ref, cnt_ref, wa_ref, ba_ref, wb_ref, bb_ref, o_ref):
    s = sum_ref[...]
    c = cnt_ref[...]
    mean = s / jnp.maximum(c, 1.0)
    z = _leaky(jnp.dot(mean, wa_ref[...], preferred_element_type=jnp.float32)
               + ba_ref[...], 0.12)
    z = jnp.dot(z, wb_ref[...], preferred_element_type=jnp.float32) + bb_ref[...]
    m = jnp.max(z, axis=1, keepdims=True)
    lse = m + jnp.log(jnp.sum(jnp.exp(z - m), axis=1, keepdims=True))
    o_ref[...] = z - lse


def _final_mlp(summed, cnt, p):
    N0 = summed.shape[0]
    return pl.pallas_call(
        _final_body,
        grid=(N0 // BN,),
        in_specs=[
            pl.BlockSpec((BN, 32), lambda i: (i, 0)),
            pl.BlockSpec((BN, 1), lambda i: (i, 0)),
            pl.BlockSpec((32, 16), lambda i: (0, 0)),
            pl.BlockSpec((1, 16), lambda i: (0, 0)),
            pl.BlockSpec((16, 2), lambda i: (0, 0)),
            pl.BlockSpec((1, 2), lambda i: (0, 0)),
        ],
        out_specs=pl.BlockSpec((BN, 2), lambda i: (i, 0)),
        out_shape=jax.ShapeDtypeStruct((N0, 2), jnp.float32),
    )(summed, cnt, p['n2a'][0], p['n2a'][1][None], p['n2b'][0], p['n2b'][1][None])


def _pad128(a):
    return jnp.pad(a, ((0, 0), (0, 128 - a.shape[1])))


def _edge_conv(hp, F, ei_flat, col, pa, pb, n):
    g = _sc_gather(hp, ei_flat, F)
    out = _ec_mlp(g, F, pa, pb)
    out = jax.ops.segment_max(out, col, num_segments=n)
    return _pad128(jnp.where(jnp.isfinite(out), out, 0.0))


def kernel(x, edge_index, e, xbatch, params):
    p = params
    n = x.shape[0]
    row, col = edge_index[0], edge_index[1]
    ei_flat = edge_index.reshape(-1)
    h = _edge_conv(_pad128(x), 16, ei_flat, col, p['ec1a'], p['ec1b'], n)
    h = _edge_conv(h, 16, ei_flat, col, p['ec2a'], p['ec2b'], n)
    h = _edge_conv(h, 32, ei_flat, col, p['ec3a'], p['ec3b'], n)
    out = _meta_mlp(_sc_gather(h, ei_flat, 64), p)
    summed = jax.ops.segment_sum(out, row, num_segments=n)
    cnt = jax.ops.segment_sum(jnp.ones((row.shape[0],), jnp.float32), row,
                              num_segments=n)
    return _final_mlp(summed, cnt[:, None], p)
